# baseline (speedup 1.0000x reference)
import functools

import jax
import jax.numpy as jnp
import numpy as np
from jax import lax
from jax.experimental import pallas as pl
from jax.experimental.pallas import tpu as pltpu
from jax.experimental.pallas import tpu_sc as plsc

_N = 4194304
_TBL = 8388608
_PAD = 128
_NC, _NS = 2, 16
_NW = _NC * _NS
_PPW = _N // _NW
_C = 8192
_NCHUNK = _PPW // _C
_NPAIR = _NCHUNK // 2
_R = _C // 128
_GPR = 8

_F_HALF = np.float32(0.5)
_F_ONE = np.float32(1.0)
_F_RES = np.float32(128.0)
_F_TOP = np.float32(1.0 - 1e-5)
_PBASE = np.int32(_TBL)
_NBW = 262144
_PBITS = np.int32(_NBW)


def _sc_body(xyz_hbm, tbl_hbm, out_hbm,
             xvA, yvA, zvA, wvA, ovA, bvA,
             xvB, yvB, zvB, wvB, ovB, bvB,
             spm, semA, semB, sxA, sxB):
    wid = lax.axis_index("s") * _NC + lax.axis_index("c")
    base = wid * _PPW

    @pl.when(lax.axis_index("s") == 0)
    def _():
        pltpu.sync_copy(tbl_hbm, spm)
    plsc.subcore_barrier()

    def start_xyz(c, xv, yv, zv, sx):
        s = base + c * _C
        pltpu.async_copy(xyz_hbm.at[pl.ds(s, _C)], xv, sx)
        pltpu.async_copy(xyz_hbm.at[pl.ds(_N + s, _C)], yv, sx)
        pltpu.async_copy(xyz_hbm.at[pl.ds(2 * _N + s, _C)], zv, sx)

    def wait_xyz(xv, yv, zv, sx):
        pltpu.make_async_copy(xyz_hbm.at[pl.ds(0, _C)], xv, sx).wait()
        pltpu.make_async_copy(xyz_hbm.at[pl.ds(0, _C)], yv, sx).wait()
        pltpu.make_async_copy(xyz_hbm.at[pl.ds(0, _C)], zv, sx).wait()

    def compute_fire(xv, yv, zv, wv, ov, bv, sem):
        def row(j, _):
            for gg in range(_GPR):
                g = j * _GPR + gg
                sl = pl.ds(g * 16, 16)
                px = xv[sl]
                py = yv[sl]
                pz = zv[sl]
                tx = (px + _F_ONE) * _F_HALF - _F_HALF
                ty = (py + _F_ONE) * _F_HALF - _F_HALF
                tz = (pz + _F_ONE) * _F_HALF - _F_HALF
                m = jnp.maximum(jnp.maximum(jnp.abs(tx), jnp.abs(ty)),
                                jnp.abs(tz))
                ebits = plsc.bitcast(m, jnp.int32) >> 23
                mip_raw = jnp.where(m > 0.0, ebits - 125, 1)
                valid = mip_raw < 4
                mip = jnp.minimum(jnp.maximum(mip_raw, 0), 3)
                scale = plsc.bitcast((127 - mip) << 23, jnp.float32)
                vx = jnp.minimum(jnp.maximum(tx * scale + _F_HALF, 0.0), _F_TOP)
                vy = jnp.minimum(jnp.maximum(ty * scale + _F_HALF, 0.0), _F_TOP)
                vz = jnp.minimum(jnp.maximum(tz * scale + _F_HALF, 0.0), _F_TOP)
                xi = (vx * _F_RES).astype(jnp.int32)
                yi = (vy * _F_RES).astype(jnp.int32)
                zi = (vz * _F_RES).astype(jnp.int32)
                idx = xi * 16384 + yi * 128 + zi + (mip << 21)
                w = ((idx >> 5) & np.int32(0x7FFF80)) | (idx & 127)
                wv[sl] = jnp.where(valid, w, _PBITS + (idx & 127))
                bv[sl] = (idx >> 7) & 31
            pltpu.async_copy(
                spm.at[wv.at[pl.ds(j * 128, 128)]],
                ov.at[pl.ds(j * 128, 128)], sem)
            return ()

        lax.fori_loop(0, _R, row, (), unroll=False)

    def drain_store(c, wv, ov, bv, sem):
        def row(j, _):
            pltpu.make_async_copy(
                spm.at[wv.at[pl.ds(j * 128, 128)]],
                ov.at[pl.ds(j * 128, 128)], sem).wait()
            return ()

        lax.fori_loop(0, _R, row, (), unroll=False)

        def ext(j, _):
            for gg in range(_GPR):
                sl = pl.ds((j * _GPR + gg) * 16, 16)
                ov[sl] = (ov[sl] >> bv[sl]) & 1
            return ()

        lax.fori_loop(0, _R, ext, (), unroll=False)
        pltpu.sync_copy(ov, out_hbm.at[pl.ds(base + c * _C, _C)])

    start_xyz(0, xvA, yvA, zvA, sxA)
    start_xyz(1, xvB, yvB, zvB, sxB)
    wait_xyz(xvA, yvA, zvA, sxA)
    compute_fire(xvA, yvA, zvA, wvA, ovA, bvA, semA)

    def pair(c2, _):
        c0 = c2 * 2
        wait_xyz(xvB, yvB, zvB, sxB)
        compute_fire(xvB, yvB, zvB, wvB, ovB, bvB, semB)

        @pl.when(c2 < _NPAIR - 1)
        def _():
            start_xyz(c0 + 2, xvA, yvA, zvA, sxA)

        drain_store(c0, wvA, ovA, bvA, semA)

        @pl.when(c2 < _NPAIR - 1)
        def _():
            wait_xyz(xvA, yvA, zvA, sxA)
            compute_fire(xvA, yvA, zvA, wvA, ovA, bvA, semA)
            start_xyz(c0 + 3, xvB, yvB, zvB, sxB)

        drain_store(c0 + 1, wvB, ovB, bvB, semB)
        return ()

    lax.fori_loop(0, _NPAIR, pair, (), unroll=False)


@jax.jit
def _occupied(xyz, tbl):
    mesh = plsc.VectorSubcoreMesh(
        core_axis_name="c", subcore_axis_name="s",
        num_cores=_NC, num_subcores=_NS)
    buf = [
        pltpu.VMEM((_C,), jnp.float32),
        pltpu.VMEM((_C,), jnp.float32),
        pltpu.VMEM((_C,), jnp.float32),
        pltpu.VMEM((_C,), jnp.int32),
        pltpu.VMEM((_C,), jnp.int32),
        pltpu.VMEM((_C,), jnp.int32),
    ]
    f = pl.kernel(
        _sc_body,
        out_type=jax.ShapeDtypeStruct((_N,), jnp.int32),
        mesh=mesh,
        compiler_params=pltpu.CompilerParams(needs_layout_passes=False),
        scratch_types=buf + buf + [
            pltpu.VMEM_SHARED((_NBW + _PAD,), jnp.int32),
            pltpu.SemaphoreType.DMA,
            pltpu.SemaphoreType.DMA,
            pltpu.SemaphoreType.DMA,
            pltpu.SemaphoreType.DMA,
        ],
    )
    return f(xyz, tbl)


def _pack_body(in_ref, out_ref):
    x = in_ref[...].astype(jnp.int32).reshape(32, 32, 128)
    q = lax.broadcasted_iota(jnp.int32, (32, 32, 128), 1)
    out_ref[...] = jnp.sum(x << q, axis=1)


def _pack_bits(occ2):
    return pl.pallas_call(
        _pack_body,
        grid=(64,),
        in_specs=[pl.BlockSpec((1024, 128), lambda i: (i, 0))],
        out_specs=pl.BlockSpec((32, 128), lambda i: (i, 0)),
        out_shape=jax.ShapeDtypeStruct((2048, 128), jnp.int32),
    )(occ2)


def kernel(pos, occs_binary, aabbs):
    xyz = pos.T.reshape(-1)
    occ2 = occs_binary.reshape(65536, 128).astype(jnp.uint8)
    tbl = jnp.pad(_pack_bits(occ2).reshape(_NBW), (0, _PAD))
    out = _occupied(xyz, tbl)
    return out.astype(jnp.bool_)


# speedup vs baseline: 3.1255x; 3.1255x over previous
import functools

import jax
import jax.numpy as jnp
import numpy as np
from jax import lax
from jax.experimental import pallas as pl
from jax.experimental.pallas import tpu as pltpu
from jax.experimental.pallas import tpu_sc as plsc

_N = 4194304
_NSLAB = 2
_SLAB = _N // _NSLAB
_TBL = 8388608
_PAD = 128
_NC, _NS = 2, 16
_NW = _NC * _NS
_PPW = _SLAB // _NW
_C = 8192
_NCHUNK = _PPW // _C
_NPAIR = _NCHUNK // 2
_R = _C // 128
_GPR = 8

_F_HALF = np.float32(0.5)
_F_ONE = np.float32(1.0)
_F_RES = np.float32(128.0)
_F_TOP = np.float32(1.0 - 1e-5)
_PBASE = np.int32(_TBL)
_NBW = 262144
_PBITS = np.int32(_NBW)


def _sc_body(x_hbm, y_hbm, z_hbm, tbl_hbm, out_hbm,
             xvA, yvA, zvA, wvA, ovA, bvA,
             xvB, yvB, zvB, wvB, ovB, bvB,
             spm, semA, semB, sxA, sxB):
    wid = lax.axis_index("s") * _NC + lax.axis_index("c")
    base = wid * _PPW

    @pl.when(lax.axis_index("s") == 0)
    def _():
        pltpu.sync_copy(tbl_hbm, spm)
    plsc.subcore_barrier()

    def start_xyz(c, xv, yv, zv, sx):
        s = base + c * _C
        pltpu.async_copy(x_hbm.at[pl.ds(s, _C)], xv, sx)
        pltpu.async_copy(y_hbm.at[pl.ds(s, _C)], yv, sx)
        pltpu.async_copy(z_hbm.at[pl.ds(s, _C)], zv, sx)

    def wait_xyz(xv, yv, zv, sx):
        pltpu.make_async_copy(x_hbm.at[pl.ds(0, _C)], xv, sx).wait()
        pltpu.make_async_copy(y_hbm.at[pl.ds(0, _C)], yv, sx).wait()
        pltpu.make_async_copy(z_hbm.at[pl.ds(0, _C)], zv, sx).wait()

    def compute_fire(xv, yv, zv, wv, ov, bv, sem):
        def row(j, _):
            for gg in range(_GPR):
                g = j * _GPR + gg
                sl = pl.ds(g * 16, 16)
                px = xv[sl]
                py = yv[sl]
                pz = zv[sl]
                tx = (px + _F_ONE) * _F_HALF - _F_HALF
                ty = (py + _F_ONE) * _F_HALF - _F_HALF
                tz = (pz + _F_ONE) * _F_HALF - _F_HALF
                m = jnp.maximum(jnp.maximum(jnp.abs(tx), jnp.abs(ty)),
                                jnp.abs(tz))
                ebits = plsc.bitcast(m, jnp.int32) >> 23
                mip_raw = jnp.where(m > 0.0, ebits - 125, 1)
                valid = mip_raw < 4
                mip = jnp.minimum(jnp.maximum(mip_raw, 0), 3)
                scale = plsc.bitcast((127 - mip) << 23, jnp.float32)
                vx = jnp.minimum(jnp.maximum(tx * scale + _F_HALF, 0.0), _F_TOP)
                vy = jnp.minimum(jnp.maximum(ty * scale + _F_HALF, 0.0), _F_TOP)
                vz = jnp.minimum(jnp.maximum(tz * scale + _F_HALF, 0.0), _F_TOP)
                xi = (vx * _F_RES).astype(jnp.int32)
                yi = (vy * _F_RES).astype(jnp.int32)
                zi = (vz * _F_RES).astype(jnp.int32)
                idx = xi * 16384 + yi * 128 + zi + (mip << 21)
                w = ((idx >> 5) & np.int32(0x7FFF80)) | (idx & 127)
                wv[sl] = jnp.where(valid, w, _PBITS + (idx & 127))
                bv[sl] = (idx >> 7) & 31
            pltpu.async_copy(
                spm.at[wv.at[pl.ds(j * 128, 128)]],
                ov.at[pl.ds(j * 128, 128)], sem)
            return ()

        lax.fori_loop(0, _R, row, (), unroll=False)

    def drain_store(c, wv, ov, bv, sem):
        def row(j, _):
            pltpu.make_async_copy(
                spm.at[wv.at[pl.ds(j * 128, 128)]],
                ov.at[pl.ds(j * 128, 128)], sem).wait()
            return ()

        lax.fori_loop(0, _R, row, (), unroll=False)

        def ext(j, _):
            for gg in range(_GPR):
                sl = pl.ds((j * _GPR + gg) * 16, 16)
                ov[sl] = (ov[sl] >> bv[sl]) & 1
            return ()

        lax.fori_loop(0, _R, ext, (), unroll=False)
        pltpu.sync_copy(ov, out_hbm.at[pl.ds(base + c * _C, _C)])

    start_xyz(0, xvA, yvA, zvA, sxA)
    start_xyz(1, xvB, yvB, zvB, sxB)
    wait_xyz(xvA, yvA, zvA, sxA)
    compute_fire(xvA, yvA, zvA, wvA, ovA, bvA, semA)

    def pair(c2, _):
        c0 = c2 * 2
        wait_xyz(xvB, yvB, zvB, sxB)
        compute_fire(xvB, yvB, zvB, wvB, ovB, bvB, semB)

        @pl.when(c2 < _NPAIR - 1)
        def _():
            start_xyz(c0 + 2, xvA, yvA, zvA, sxA)

        drain_store(c0, wvA, ovA, bvA, semA)

        @pl.when(c2 < _NPAIR - 1)
        def _():
            wait_xyz(xvA, yvA, zvA, sxA)
            compute_fire(xvA, yvA, zvA, wvA, ovA, bvA, semA)
            start_xyz(c0 + 3, xvB, yvB, zvB, sxB)

        drain_store(c0 + 1, wvB, ovB, bvB, semB)
        return ()

    lax.fori_loop(0, _NPAIR, pair, (), unroll=False)


@jax.jit
def _occupied(x, y, z, tbl):
    mesh = plsc.VectorSubcoreMesh(
        core_axis_name="c", subcore_axis_name="s",
        num_cores=_NC, num_subcores=_NS)
    buf = [
        pltpu.VMEM((_C,), jnp.float32),
        pltpu.VMEM((_C,), jnp.float32),
        pltpu.VMEM((_C,), jnp.float32),
        pltpu.VMEM((_C,), jnp.int32),
        pltpu.VMEM((_C,), jnp.int32),
        pltpu.VMEM((_C,), jnp.int32),
    ]
    f = pl.kernel(
        _sc_body,
        out_type=jax.ShapeDtypeStruct((_SLAB,), jnp.int32),
        mesh=mesh,
        compiler_params=pltpu.CompilerParams(needs_layout_passes=False),
        cost_estimate=pl.CostEstimate(
            flops=60 * _SLAB, transcendentals=0,
            bytes_accessed=12 * _SLAB + 4 * _SLAB + 4 * _SLAB),
        scratch_types=buf + buf + [
            pltpu.VMEM_SHARED((_NBW + _PAD,), jnp.int32),
            pltpu.SemaphoreType.DMA,
            pltpu.SemaphoreType.DMA,
            pltpu.SemaphoreType.DMA,
            pltpu.SemaphoreType.DMA,
        ],
    )
    return f(x, y, z, tbl)


def _pack_body(in_ref, out_ref):
    x = in_ref[...].astype(jnp.int32).reshape(32, 32, 128)
    q = lax.broadcasted_iota(jnp.int32, (32, 32, 128), 1)
    out_ref[...] = jnp.sum(x << q, axis=1)


def _pack_bits(occ2):
    return pl.pallas_call(
        _pack_body,
        grid=(64,),
        in_specs=[pl.BlockSpec((1024, 128), lambda i: (i, 0))],
        out_specs=pl.BlockSpec((32, 128), lambda i: (i, 0)),
        out_shape=jax.ShapeDtypeStruct((2048, 128), jnp.int32),
    )(occ2)


def kernel(pos, occs_binary, aabbs):
    occ2 = occs_binary.reshape(65536, 128).astype(jnp.uint8)
    tbl = jnp.pad(_pack_bits(occ2).reshape(_NBW), (0, _PAD))
    outs = []
    for sl in range(_NSLAB):
        lo = sl * _SLAB
        x = lax.slice(pos, (lo, 0), (lo + _SLAB, 1)).reshape(_SLAB)
        y = lax.slice(pos, (lo, 1), (lo + _SLAB, 2)).reshape(_SLAB)
        z = lax.slice(pos, (lo, 2), (lo + _SLAB, 3)).reshape(_SLAB)
        outs.append(_occupied(x, y, z, tbl).astype(jnp.bool_))
    return jnp.concatenate(outs)


# speedup vs baseline: 3.9491x; 1.2635x over previous
import functools

import jax
import jax.numpy as jnp
import numpy as np
from jax import lax
from jax.experimental import pallas as pl
from jax.experimental.pallas import tpu as pltpu
from jax.experimental.pallas import tpu_sc as plsc

_N = 4194304
_TBL = 8388608
_PAD = 128
_NC, _NS = 2, 16
_NW = _NC * _NS
_PPW = _N // _NW
_C = 8192
_NCHUNK = _PPW // _C
_NPAIR = _NCHUNK // 2
_R = _C // 128
_GPR = 8

_F_HALF = np.float32(0.5)
_F_ONE = np.float32(1.0)
_F_RES = np.float32(128.0)
_F_TOP = np.float32(1.0 - 1e-5)
_PBASE = np.int32(_TBL)
_NBW = 262144
_PBITS = np.int32(_NBW)


def _sc_body(x_hbm, y_hbm, z_hbm, tbl_hbm, out_hbm,
             xvA, yvA, zvA, wvA, ovA, bvA,
             xvB, yvB, zvB, wvB, ovB, bvB,
             spm, semA, semB, sxA, sxB):
    wid = lax.axis_index("s") * _NC + lax.axis_index("c")
    base = wid * _PPW

    @pl.when(lax.axis_index("s") == 0)
    def _():
        pltpu.sync_copy(tbl_hbm, spm)
    plsc.subcore_barrier()

    def start_xyz(c, xv, yv, zv, sx):
        s = base + c * _C
        pltpu.async_copy(x_hbm.at[pl.ds(s, _C)], xv, sx)
        pltpu.async_copy(y_hbm.at[pl.ds(s, _C)], yv, sx)
        pltpu.async_copy(z_hbm.at[pl.ds(s, _C)], zv, sx)

    def wait_xyz(xv, yv, zv, sx):
        pltpu.make_async_copy(x_hbm.at[pl.ds(0, _C)], xv, sx).wait()
        pltpu.make_async_copy(y_hbm.at[pl.ds(0, _C)], yv, sx).wait()
        pltpu.make_async_copy(z_hbm.at[pl.ds(0, _C)], zv, sx).wait()

    def compute_fire(xv, yv, zv, wv, ov, bv, sem):
        def row(j, _):
            for gg in range(_GPR):
                g = j * _GPR + gg
                sl = pl.ds(g * 16, 16)
                px = xv[sl]
                py = yv[sl]
                pz = zv[sl]
                tx = (px + _F_ONE) * _F_HALF - _F_HALF
                ty = (py + _F_ONE) * _F_HALF - _F_HALF
                tz = (pz + _F_ONE) * _F_HALF - _F_HALF
                m = jnp.maximum(jnp.maximum(jnp.abs(tx), jnp.abs(ty)),
                                jnp.abs(tz))
                ebits = plsc.bitcast(m, jnp.int32) >> 23
                mip_raw = jnp.where(m > 0.0, ebits - 125, 1)
                valid = mip_raw < 4
                mip = jnp.minimum(jnp.maximum(mip_raw, 0), 3)
                scale = plsc.bitcast((127 - mip) << 23, jnp.float32)
                vx = jnp.minimum(jnp.maximum(tx * scale + _F_HALF, 0.0), _F_TOP)
                vy = jnp.minimum(jnp.maximum(ty * scale + _F_HALF, 0.0), _F_TOP)
                vz = jnp.minimum(jnp.maximum(tz * scale + _F_HALF, 0.0), _F_TOP)
                xi = (vx * _F_RES).astype(jnp.int32)
                yi = (vy * _F_RES).astype(jnp.int32)
                zi = (vz * _F_RES).astype(jnp.int32)
                idx = xi * 16384 + yi * 128 + zi + (mip << 21)
                w = ((idx >> 5) & np.int32(0x7FFF80)) | (idx & 127)
                wv[sl] = jnp.where(valid, w, _PBITS + (idx & 127))
                bv[sl] = (idx >> 7) & 31
            pltpu.async_copy(
                spm.at[wv.at[pl.ds(j * 128, 128)]],
                ov.at[pl.ds(j * 128, 128)], sem)
            return ()

        lax.fori_loop(0, _R, row, (), unroll=False)

    def drain_store(c, wv, ov, bv, sem):
        def row(j, _):
            pltpu.make_async_copy(
                spm.at[wv.at[pl.ds(j * 128, 128)]],
                ov.at[pl.ds(j * 128, 128)], sem).wait()
            return ()

        lax.fori_loop(0, _R, row, (), unroll=False)

        def ext(j, _):
            for gg in range(_GPR):
                sl = pl.ds((j * _GPR + gg) * 16, 16)
                ov[sl] = (ov[sl] >> bv[sl]) & 1
            return ()

        lax.fori_loop(0, _R, ext, (), unroll=False)
        pltpu.sync_copy(ov, out_hbm.at[pl.ds(base + c * _C, _C)])

    start_xyz(0, xvA, yvA, zvA, sxA)
    start_xyz(1, xvB, yvB, zvB, sxB)
    wait_xyz(xvA, yvA, zvA, sxA)
    compute_fire(xvA, yvA, zvA, wvA, ovA, bvA, semA)

    def pair(c2, _):
        c0 = c2 * 2
        wait_xyz(xvB, yvB, zvB, sxB)
        compute_fire(xvB, yvB, zvB, wvB, ovB, bvB, semB)

        @pl.when(c2 < _NPAIR - 1)
        def _():
            start_xyz(c0 + 2, xvA, yvA, zvA, sxA)

        drain_store(c0, wvA, ovA, bvA, semA)

        @pl.when(c2 < _NPAIR - 1)
        def _():
            wait_xyz(xvA, yvA, zvA, sxA)
            compute_fire(xvA, yvA, zvA, wvA, ovA, bvA, semA)
            start_xyz(c0 + 3, xvB, yvB, zvB, sxB)

        drain_store(c0 + 1, wvB, ovB, bvB, semB)
        return ()

    lax.fori_loop(0, _NPAIR, pair, (), unroll=False)


@jax.jit
def _occupied(x, y, z, tbl):
    mesh = plsc.VectorSubcoreMesh(
        core_axis_name="c", subcore_axis_name="s",
        num_cores=_NC, num_subcores=_NS)
    buf = [
        pltpu.VMEM((_C,), jnp.float32),
        pltpu.VMEM((_C,), jnp.float32),
        pltpu.VMEM((_C,), jnp.float32),
        pltpu.VMEM((_C,), jnp.int32),
        pltpu.VMEM((_C,), jnp.int32),
        pltpu.VMEM((_C,), jnp.int32),
    ]
    f = pl.kernel(
        _sc_body,
        out_type=jax.ShapeDtypeStruct((_N,), jnp.int32),
        mesh=mesh,
        compiler_params=pltpu.CompilerParams(needs_layout_passes=False),
        scratch_types=buf + buf + [
            pltpu.VMEM_SHARED((_NBW + _PAD,), jnp.int32),
            pltpu.SemaphoreType.DMA,
            pltpu.SemaphoreType.DMA,
            pltpu.SemaphoreType.DMA,
            pltpu.SemaphoreType.DMA,
        ],
    )
    return f(x, y, z, tbl)


def kernel(pos, occs_binary, aabbs):
    x = pos[:, 0]
    y = pos[:, 1]
    z = pos[:, 2]
    occ3 = occs_binary.reshape(2048, 32, 128).astype(jnp.int32)
    q = lax.broadcasted_iota(jnp.int32, (2048, 32, 128), 1)
    tbl = jnp.pad((occ3 << q).sum(axis=1).reshape(_NBW), (0, _PAD))
    out = _occupied(x, y, z, tbl)
    return out.astype(jnp.bool_)
